# Initial kernel scaffold; baseline (speedup 1.0000x reference)
import jax
import jax.numpy as jnp
from jax.experimental import pallas as pl


def kernel(x, sample):
    raise NotImplementedError("write your pallas kernel here")


# speedup vs baseline: 2.4541x; 2.4541x over previous
import functools

import jax
import jax.numpy as jnp
from jax import lax
from jax.experimental import pallas as pl
from jax.experimental.pallas import tpu as pltpu
from jax.experimental.pallas import tpu_sc as plsc

B, N, D = 4, 4096, 1024
ROWS = B * N
NW = 32
RPW = ROWS // NW
C = 32
NCH = RPW // C

_mesh = plsc.VectorSubcoreMesh(core_axis_name="c", subcore_axis_name="s")


@functools.partial(
    pl.kernel,
    mesh=_mesh,
    out_type=jax.ShapeDtypeStruct((NW, NCH, C, D), jnp.float32),
    scratch_types=[
        pltpu.VMEM((NCH, C), jnp.int32),
        pltpu.VMEM((C, D), jnp.float32),
        pltpu.VMEM((C, D), jnp.float32),
        pltpu.SemaphoreType.DMA,
        pltpu.SemaphoreType.DMA,
    ],
)
def _shuffle_sc(x_hbm, idx_hbm, out_hbm, idx_v, buf0, buf1, sem0, sem1):
    wid = lax.axis_index("s") * 2 + lax.axis_index("c")
    pltpu.sync_copy(idx_hbm.at[wid], idx_v)
    bufs = (buf0, buf1)
    sems = (sem0, sem1)
    handles = [None] * NCH
    handles[0] = pltpu.async_copy(x_hbm.at[idx_v.at[0]], bufs[0], sems[0])
    for j in range(NCH):
        slot = j % 2
        if j + 1 < NCH:
            nslot = (j + 1) % 2
            handles[j + 1] = pltpu.async_copy(
                x_hbm.at[idx_v.at[j + 1]], bufs[nslot], sems[nslot])
        handles[j].wait()
        pltpu.sync_copy(bufs[slot], out_hbm.at[wid, j])


def kernel(x, sample):
    b, n, d = x.shape
    perm = jax.random.permutation(jax.random.key(42), n).astype(jnp.int32)
    base = (jnp.arange(b, dtype=jnp.int32) * n)[:, None]
    idx_sh = base + perm[None, :]
    idx_id = base + jnp.arange(n, dtype=jnp.int32)[None, :]
    idx = jnp.where(sample != 0, idx_sh, idx_id).reshape(NW, NCH, C)
    out = _shuffle_sc(x.reshape(b * n, d), idx)
    return out.reshape(b, n, d)


# speedup vs baseline: 2.4724x; 1.0075x over previous
import functools

import jax
import jax.numpy as jnp
from jax import lax
from jax.experimental import pallas as pl
from jax.experimental.pallas import tpu as pltpu
from jax.experimental.pallas import tpu_sc as plsc

B, N, D = 4, 4096, 1024
ROWS = B * N
NW = 32
RPW = ROWS // NW
C = 32
NCH = RPW // C

K = 3

_mesh = plsc.VectorSubcoreMesh(core_axis_name="c", subcore_axis_name="s")


@functools.partial(
    pl.kernel,
    mesh=_mesh,
    out_type=jax.ShapeDtypeStruct((NW, NCH, C, D), jnp.float32),
    scratch_types=[
        pltpu.VMEM((NCH, C), jnp.int32),
        pltpu.VMEM((C, D), jnp.float32),
        pltpu.VMEM((C, D), jnp.float32),
        pltpu.VMEM((C, D), jnp.float32),
        pltpu.SemaphoreType.DMA,
        pltpu.SemaphoreType.DMA,
        pltpu.SemaphoreType.DMA,
        pltpu.SemaphoreType.DMA,
        pltpu.SemaphoreType.DMA,
        pltpu.SemaphoreType.DMA,
    ],
)
def _shuffle_sc(x_hbm, idx_hbm, out_hbm, idx_v, b0, b1, b2,
                gs0, gs1, gs2, ws0, ws1, ws2):
    wid = lax.axis_index("s") * 2 + lax.axis_index("c")
    pltpu.sync_copy(idx_hbm.at[wid], idx_v)
    bufs = (b0, b1, b2)
    gsems = (gs0, gs1, gs2)
    wsems = (ws0, ws1, ws2)
    g = [None] * NCH
    w = [None] * NCH
    for j in range(K):
        g[j] = pltpu.async_copy(x_hbm.at[idx_v.at[j]], bufs[j], gsems[j])
    for j in range(NCH):
        s = j % K
        if j >= 1 and j - 1 + K < NCH:
            p = j - 1
            w[p].wait()
            g[p + K] = pltpu.async_copy(
                x_hbm.at[idx_v.at[p + K]], bufs[p % K], gsems[p % K])
        g[j].wait()
        w[j] = pltpu.async_copy(bufs[s], out_hbm.at[wid, j], wsems[s])
    for j in range(NCH - K, NCH):
        if w[j] is not None:
            w[j].wait()


def kernel(x, sample):
    b, n, d = x.shape
    perm = jax.random.permutation(jax.random.key(42), n).astype(jnp.int32)
    base = (jnp.arange(b, dtype=jnp.int32) * n)[:, None]
    idx_sh = base + perm[None, :]
    idx_id = base + jnp.arange(n, dtype=jnp.int32)[None, :]
    idx = jnp.where(sample != 0, idx_sh, idx_id).reshape(NW, NCH, C)
    out = _shuffle_sc(x.reshape(b * n, d), idx)
    return out.reshape(b, n, d)


# speedup vs baseline: 3.4231x; 1.3845x over previous
import functools

import numpy as np

import jax
import jax.numpy as jnp
from jax import lax
from jax.experimental import pallas as pl
from jax.experimental.pallas import tpu as pltpu
from jax.experimental.pallas import tpu_sc as plsc

B, N, D = 4, 4096, 1024
ROWS = B * N
NW = 32
RPW = ROWS // NW
C = 32
NCH = RPW // C

K = 3

_mesh = plsc.VectorSubcoreMesh(core_axis_name="c", subcore_axis_name="s")


@functools.partial(
    pl.kernel,
    mesh=_mesh,
    out_type=jax.ShapeDtypeStruct((NW, NCH, C, D), jnp.float32),
    scratch_types=[
        pltpu.VMEM((NCH, C), jnp.int32),
        pltpu.VMEM((C, D), jnp.float32),
        pltpu.VMEM((C, D), jnp.float32),
        pltpu.VMEM((C, D), jnp.float32),
        pltpu.SemaphoreType.DMA,
        pltpu.SemaphoreType.DMA,
        pltpu.SemaphoreType.DMA,
        pltpu.SemaphoreType.DMA,
        pltpu.SemaphoreType.DMA,
        pltpu.SemaphoreType.DMA,
    ],
)
def _shuffle_sc(x_hbm, idx_hbm, out_hbm, idx_v, b0, b1, b2,
                gs0, gs1, gs2, ws0, ws1, ws2):
    wid = lax.axis_index("s") * 2 + lax.axis_index("c")
    pltpu.sync_copy(idx_hbm.at[wid], idx_v)
    bufs = (b0, b1, b2)
    gsems = (gs0, gs1, gs2)
    wsems = (ws0, ws1, ws2)
    g = [None] * NCH
    w = [None] * NCH
    for j in range(K):
        g[j] = pltpu.async_copy(x_hbm.at[idx_v.at[j]], bufs[j], gsems[j])
    for j in range(NCH):
        s = j % K
        if j >= 1 and j - 1 + K < NCH:
            p = j - 1
            w[p].wait()
            g[p + K] = pltpu.async_copy(
                x_hbm.at[idx_v.at[p + K]], bufs[p % K], gsems[p % K])
        g[j].wait()
        w[j] = pltpu.async_copy(bufs[s], out_hbm.at[wid, j], wsems[s])
    for j in range(NCH - K, NCH):
        if w[j] is not None:
            w[j].wait()


_IDX_CACHE = {}


def _flat_indices(n):
    if n not in _IDX_CACHE:
        with jax.ensure_compile_time_eval():
            perm = np.asarray(
                jax.random.permutation(jax.random.key(42), n)).astype(np.int32)
        base = (np.arange(B, dtype=np.int32) * n)[:, None]
        idx_sh = (base + perm[None, :]).reshape(NW, NCH, C)
        idx_id = (base + np.arange(n, dtype=np.int32)[None, :]).reshape(
            NW, NCH, C)
        _IDX_CACHE[n] = (idx_sh, idx_id)
    return _IDX_CACHE[n]


def kernel(x, sample):
    b, n, d = x.shape
    idx_sh, idx_id = _flat_indices(n)
    idx = jnp.where(sample != 0, jnp.asarray(idx_sh), jnp.asarray(idx_id))
    out = _shuffle_sc(x.reshape(b * n, d), idx)
    return out.reshape(b, n, d)
